# Initial kernel scaffold; baseline (speedup 1.0000x reference)
import jax
import jax.numpy as jnp
from jax.experimental import pallas as pl


def kernel(tokens, lengths, starts, ends):
    raise NotImplementedError("write your pallas kernel here")


# speedup vs baseline: 1.5639x; 1.5639x over previous
import jax
import jax.numpy as jnp
from jax import lax
from jax.experimental import pallas as pl
from jax.experimental.pallas import tpu as pltpu
from jax.experimental.pallas import tpu_sc as plsc

B = 16
SEG = 2
L = 512
M = 510
OUT = M + 3
LANES = 16
NCHUNK = 33
OUT_PAD = NCHUNK * LANES
CLS_ID = 2
SEP_ID = 3
NC = 2
NS = 16


def _body(tok_h, len_h, st_h, en_h,
          tokid_h, typ_h, sto_h, eno_h,
          tok_v, len_v, tid_v, typ_v, st_v, en_v, sto_v, eno_v):
  wid = lax.axis_index("s") * NC + lax.axis_index("c")
  b = wid % B
  task = wid // B
  iota = lax.iota(jnp.int32, LANES)
  zeros = jnp.zeros((LANES,), jnp.int32)
  bvec = jnp.broadcast_to(b, (LANES,)).astype(jnp.int32)

  @pl.when(task == 0)
  def _tokens():
    pltpu.sync_copy(tok_h.at[b], tok_v)
    pltpu.sync_copy(len_h, len_v)
    l1 = plsc.load_gather(len_v, [bvec * 2])
    l2 = plsc.load_gather(len_v, [bvec * 2 + 1])
    k1 = jnp.minimum(l1, jnp.maximum((M + 1) // 2, M - l2))
    k2 = jnp.minimum(l2, jnp.maximum(M // 2, M - l1))
    for c in range(NCHUNK):
      pos = iota + (c * LANES)
      idx1 = jnp.clip(pos - 1, 0, L - 1)
      idx2 = jnp.clip(pos - k1 - 2, 0, L - 1)
      t1 = plsc.load_gather(tok_v, [idx1])
      t2 = plsc.load_gather(tok_v, [idx2 + L])
      val = jnp.where(pos == 0, CLS_ID, 0).astype(jnp.int32)
      val = jnp.where((pos >= 1) & (pos <= k1), t1, val)
      val = jnp.where(pos == k1 + 1, SEP_ID, val)
      val = jnp.where((pos >= k1 + 2) & (pos <= k1 + k2 + 1), t2, val)
      val = jnp.where(pos == k1 + k2 + 2, SEP_ID, val)
      typ = jnp.where((pos >= k1 + 2) & (pos <= k1 + k2 + 2), 1, 0)
      sl = pl.ds(c * LANES, LANES)
      tid_v[sl] = val
      typ_v[sl] = typ.astype(jnp.int32)
    pltpu.sync_copy(tid_v, tokid_h.at[b])
    pltpu.sync_copy(typ_v, typ_h.at[b])

  @pl.when(task == 1)
  def _offsets():
    pltpu.sync_copy(st_h.at[b], st_v)
    pltpu.sync_copy(en_h.at[b], en_v)
    pltpu.sync_copy(len_h, len_v)
    l1 = plsc.load_gather(len_v, [bvec * 2])
    for c in range(L // LANES):
      col = iota + (c * LANES)
      m = col < l1
      sl = pl.ds(c * LANES, LANES)
      sto_v[sl] = jnp.where(m, st_v[sl], 0.0)
      eno_v[sl] = jnp.where(m, en_v[sl], 0.0)
    pltpu.sync_copy(sto_v, sto_h.at[b])
    pltpu.sync_copy(eno_v, eno_h.at[b])


@jax.jit
def kernel(tokens, lengths, starts, ends):
  mesh = plsc.VectorSubcoreMesh(
      core_axis_name="c", subcore_axis_name="s",
      num_cores=NC, num_subcores=NS)
  run = pl.kernel(
      _body,
      out_type=(
          jax.ShapeDtypeStruct((B, OUT_PAD), jnp.int32),
          jax.ShapeDtypeStruct((B, OUT_PAD), jnp.int32),
          jax.ShapeDtypeStruct((B, L), jnp.float32),
          jax.ShapeDtypeStruct((B, L), jnp.float32),
      ),
      mesh=mesh,
      compiler_params=pltpu.CompilerParams(needs_layout_passes=False),
      scratch_types=[
          pltpu.VMEM((SEG * L,), jnp.int32),
          pltpu.VMEM((B * SEG,), jnp.int32),
          pltpu.VMEM((OUT_PAD,), jnp.int32),
          pltpu.VMEM((OUT_PAD,), jnp.int32),
          pltpu.VMEM((L,), jnp.float32),
          pltpu.VMEM((L,), jnp.float32),
          pltpu.VMEM((L,), jnp.float32),
          pltpu.VMEM((L,), jnp.float32),
      ],
  )
  tokid, typ, sto, eno = run(
      tokens.reshape(B, SEG * L), lengths.reshape(B * SEG), starts, ends)
  return tokid[:, :OUT], typ[:, :OUT], sto, eno


# speedup vs baseline: 1.6588x; 1.0607x over previous
import jax
import jax.numpy as jnp
from jax import lax
from jax.experimental import pallas as pl
from jax.experimental.pallas import tpu as pltpu
from jax.experimental.pallas import tpu_sc as plsc

B = 16
SEG = 2
L = 512
M = 510
OUT = M + 3
LANES = 16
NCHUNK = 33
OUT_PAD = NCHUNK * LANES
CLS_ID = 2
SEP_ID = 3
NC = 2
NS = 16


def _body(tok_h, len_h, st_h, en_h,
          tokid_h, typ_h, sto_h, eno_h,
          tok_v, len_v, tid_v, typ_v, st_v, en_v, sto_v, eno_v,
          sem0, sem1, sem2):
  wid = lax.axis_index("s") * NC + lax.axis_index("c")
  b = wid % B
  task = wid // B
  iota = lax.iota(jnp.int32, LANES)
  zeros = jnp.zeros((LANES,), jnp.int32)
  bvec = jnp.broadcast_to(b, (LANES,)).astype(jnp.int32)

  @pl.when(task == 0)
  def _tokens():
    cp_tok0 = pltpu.async_copy(tok_h.at[b, 0], tok_v.at[pl.ds(0, L)], sem0)
    cp_tok1 = pltpu.async_copy(tok_h.at[b, 1], tok_v.at[pl.ds(L, L)], sem2)
    cp_len = pltpu.async_copy(len_h, len_v, sem1)
    cp_len.wait()
    l1 = plsc.load_gather(len_v, [bvec * 2])
    l2 = plsc.load_gather(len_v, [bvec * 2 + 1])
    k1 = jnp.minimum(l1, jnp.maximum((M + 1) // 2, M - l2))
    k2 = jnp.minimum(l2, jnp.maximum(M // 2, M - l1))
    cp_tok0.wait()
    cp_tok1.wait()

    def chunk(c, carry):
      pos = iota + c * LANES
      idx1 = jnp.clip(pos - 1, 0, L - 1)
      idx2 = jnp.clip(pos - k1 - 2, 0, L - 1)
      t1 = plsc.load_gather(tok_v, [idx1])
      t2 = plsc.load_gather(tok_v, [idx2 + L])
      val = jnp.where(pos == 0, CLS_ID, 0).astype(jnp.int32)
      val = jnp.where((pos >= 1) & (pos <= k1), t1, val)
      val = jnp.where(pos == k1 + 1, SEP_ID, val)
      val = jnp.where((pos >= k1 + 2) & (pos <= k1 + k2 + 1), t2, val)
      val = jnp.where(pos == k1 + k2 + 2, SEP_ID, val)
      typ = jnp.where((pos >= k1 + 2) & (pos <= k1 + k2 + 2), 1, 0)
      sl = pl.ds(c * LANES, LANES)
      tid_v[sl] = val
      typ_v[sl] = typ.astype(jnp.int32)
      return carry

    lax.fori_loop(0, NCHUNK, chunk, 0, unroll=False)
    cp_o0 = pltpu.async_copy(tid_v, tokid_h.at[b], sem0)
    cp_o1 = pltpu.async_copy(typ_v, typ_h.at[b], sem1)
    cp_o0.wait()
    cp_o1.wait()

  @pl.when(task == 1)
  def _offsets():
    cp_st = pltpu.async_copy(st_h.at[b], st_v, sem0)
    cp_en = pltpu.async_copy(en_h.at[b], en_v, sem1)
    cp_len = pltpu.async_copy(len_h, len_v, sem2)
    cp_len.wait()
    l1 = plsc.load_gather(len_v, [bvec * 2])
    cp_st.wait()
    cp_en.wait()
    def ochunk(c, carry):
      col = iota + c * LANES
      m = col < l1
      sl = pl.ds(c * LANES, LANES)
      sto_v[sl] = jnp.where(m, st_v[sl], 0.0)
      eno_v[sl] = jnp.where(m, en_v[sl], 0.0)
      return carry

    lax.fori_loop(0, L // LANES, ochunk, 0, unroll=False)
    cp_o0 = pltpu.async_copy(sto_v, sto_h.at[b], sem0)
    cp_o1 = pltpu.async_copy(eno_v, eno_h.at[b], sem1)
    cp_o0.wait()
    cp_o1.wait()


@jax.jit
def kernel(tokens, lengths, starts, ends):
  mesh = plsc.VectorSubcoreMesh(
      core_axis_name="c", subcore_axis_name="s",
      num_cores=NC, num_subcores=NS)
  run = pl.kernel(
      _body,
      out_type=(
          jax.ShapeDtypeStruct((B, OUT_PAD), jnp.int32),
          jax.ShapeDtypeStruct((B, OUT_PAD), jnp.int32),
          jax.ShapeDtypeStruct((B, L), jnp.float32),
          jax.ShapeDtypeStruct((B, L), jnp.float32),
      ),
      mesh=mesh,
      compiler_params=pltpu.CompilerParams(needs_layout_passes=False),
      scratch_types=[
          pltpu.VMEM((SEG * L,), jnp.int32),
          pltpu.VMEM((B * SEG,), jnp.int32),
          pltpu.VMEM((OUT_PAD,), jnp.int32),
          pltpu.VMEM((OUT_PAD,), jnp.int32),
          pltpu.VMEM((L,), jnp.float32),
          pltpu.VMEM((L,), jnp.float32),
          pltpu.VMEM((L,), jnp.float32),
          pltpu.VMEM((L,), jnp.float32),
          pltpu.SemaphoreType.DMA,
          pltpu.SemaphoreType.DMA,
          pltpu.SemaphoreType.DMA,
      ],
  )
  tokid, typ, sto, eno = run(tokens, lengths.reshape(B * SEG), starts, ends)
  return tokid[:, :OUT], typ[:, :OUT], sto, eno
